# baseline (speedup 1.0000x reference)
import jax
import jax.numpy as jnp
from jax.experimental import pallas as pl
from jax.experimental.pallas import tpu as pltpu

HIDDEN_DIM = 2048
N_EXPERTS = 16
K = 2


def _router_kernel(x_ref, w_ref, logits_ref, probs_ref, idx_ref):
    x = x_ref[...]
    w = w_ref[...]
    tile = x.shape[0]
    lt = jax.lax.dot_general(
        w, x, (((1,), (1,)), ((), ())),
        preferred_element_type=jnp.float32,
    )
    logits_ref[...] = lt

    erows = jax.lax.broadcasted_iota(jnp.int32, lt.shape, 0)
    l1 = jnp.max(lt, axis=0)
    i1 = jnp.argmax(lt, axis=0)
    masked = jnp.where(erows == i1[None, :], -jnp.inf, lt)
    l2 = jnp.max(masked, axis=0)
    i2 = jnp.argmax(masked, axis=0)

    e2 = jnp.exp(l2 - l1)
    r = 1.0 / (1.0 + e2)
    krows = jax.lax.broadcasted_iota(jnp.int32, (K, tile), 0)
    probs_ref[...] = jnp.where(krows == 0, r[None, :], (e2 * r)[None, :])
    idx_ref[...] = jnp.where(krows == 0, i1[None, :], i2[None, :])


def kernel(hidden_states, gate_weight):
    B, S, H = hidden_states.shape
    T = B * S
    x = hidden_states.reshape(T, H)

    TILE = 512
    grid = (T // TILE,)

    logits_t, probs_t, idx_t = pl.pallas_call(
        _router_kernel,
        grid=grid,
        in_specs=[
            pl.BlockSpec((TILE, H), lambda i: (i, 0)),
            pl.BlockSpec((N_EXPERTS, H), lambda i: (0, 0)),
        ],
        out_specs=[
            pl.BlockSpec((N_EXPERTS, TILE), lambda i: (0, i)),
            pl.BlockSpec((K, TILE), lambda i: (0, i)),
            pl.BlockSpec((K, TILE), lambda i: (0, i)),
        ],
        out_shape=[
            jax.ShapeDtypeStruct((N_EXPERTS, T), jnp.float32),
            jax.ShapeDtypeStruct((K, T), jnp.float32),
            jax.ShapeDtypeStruct((K, T), jnp.int32),
        ],
        compiler_params=pltpu.CompilerParams(
            dimension_semantics=("parallel",),
        ),
    )(x, gate_weight.astype(hidden_states.dtype))

    return (
        probs_t.T.reshape(B, S, K),
        idx_t.T.reshape(B, S, K),
        logits_t.T.reshape(B, S, N_EXPERTS),
    )


# speedup vs baseline: 1.1556x; 1.1556x over previous
import jax
import jax.numpy as jnp
from jax.experimental import pallas as pl
from jax.experimental.pallas import tpu as pltpu

HIDDEN_DIM = 2048
N_EXPERTS = 16
K = 2


def _router_kernel(xa_ref, xb_ref, w_ref, logits_ref, probs_ref, idx_ref):
    w = w_ref[...]
    tile = xa_ref.shape[0] + xb_ref.shape[0]
    lt = jnp.concatenate([
        jax.lax.dot_general(w, xa_ref[...], (((1,), (1,)), ((), ())),
                            preferred_element_type=jnp.float32),
        jax.lax.dot_general(w, xb_ref[...], (((1,), (1,)), ((), ())),
                            preferred_element_type=jnp.float32),
    ], axis=1)
    logits_ref[...] = lt

    erows = jax.lax.broadcasted_iota(jnp.int32, lt.shape, 0)
    l1 = jnp.max(lt, axis=0)
    i1 = jnp.argmax(lt, axis=0)
    masked = jnp.where(erows == i1[None, :], -jnp.inf, lt)
    l2 = jnp.max(masked, axis=0)
    i2 = jnp.argmax(masked, axis=0)

    e2 = jnp.exp(l2 - l1)
    r = 1.0 / (1.0 + e2)
    krows = jax.lax.broadcasted_iota(jnp.int32, (K, tile), 0)
    probs_ref[...] = jnp.where(krows == 0, r[None, :], (e2 * r)[None, :])
    idx_ref[...] = jnp.where(krows == 0, i1[None, :], i2[None, :])


def kernel(hidden_states, gate_weight):
    B, S, H = hidden_states.shape
    T = B * S
    x = hidden_states.reshape(T, H)

    TILE = 1024
    grid = (T // TILE,)

    logits_t, probs_t, idx_t = pl.pallas_call(
        _router_kernel,
        grid=grid,
        in_specs=[
            pl.BlockSpec((TILE // 2, H), lambda i: (2 * i, 0)),
            pl.BlockSpec((TILE // 2, H), lambda i: (2 * i + 1, 0)),
            pl.BlockSpec((N_EXPERTS, H), lambda i: (0, 0)),
        ],
        out_specs=[
            pl.BlockSpec((N_EXPERTS, TILE), lambda i: (0, i)),
            pl.BlockSpec((K, TILE), lambda i: (0, i)),
            pl.BlockSpec((K, TILE), lambda i: (0, i)),
        ],
        out_shape=[
            jax.ShapeDtypeStruct((N_EXPERTS, T), jnp.float32),
            jax.ShapeDtypeStruct((K, T), jnp.float32),
            jax.ShapeDtypeStruct((K, T), jnp.int32),
        ],
        compiler_params=pltpu.CompilerParams(
            dimension_semantics=("parallel",),
        ),
    )(x, x, gate_weight.astype(hidden_states.dtype))

    return (
        probs_t.T.reshape(B, S, K),
        idx_t.T.reshape(B, S, K),
        logits_t.T.reshape(B, S, N_EXPERTS),
    )


# speedup vs baseline: 1.1763x; 1.0180x over previous
import jax
import jax.numpy as jnp
from jax.experimental import pallas as pl
from jax.experimental.pallas import tpu as pltpu

HIDDEN_DIM = 2048
N_EXPERTS = 16
K = 2


def _router_kernel(x_ref, w_ref, logits_ref, probs_ref, idx_ref):
    x = x_ref[...]
    w = w_ref[...]
    tile = x.shape[0]
    lt = jax.lax.dot_general(
        w, x, (((1,), (1,)), ((), ())),
        preferred_element_type=jnp.float32,
    )
    logits_ref[...] = lt

    erows = jax.lax.broadcasted_iota(jnp.int32, lt.shape, 0)
    l1 = jnp.max(lt, axis=0)
    i1 = jnp.argmax(lt, axis=0)
    masked = jnp.where(erows == i1[None, :], -jnp.inf, lt)
    l2 = jnp.max(masked, axis=0)
    i2 = jnp.argmax(masked, axis=0)

    e2 = jnp.exp(l2 - l1)
    r = 1.0 / (1.0 + e2)
    krows = jax.lax.broadcasted_iota(jnp.int32, (K, tile), 0)
    probs_ref[...] = jnp.where(krows == 0, r[None, :], (e2 * r)[None, :])
    idx_ref[...] = jnp.where(krows == 0, i1[None, :], i2[None, :])


def kernel(hidden_states, gate_weight):
    B, S, H = hidden_states.shape
    T = B * S
    x = hidden_states.reshape(T, H)

    TILE = 1024
    grid = (T // TILE,)

    logits_t, probs_t, idx_t = pl.pallas_call(
        _router_kernel,
        grid=grid,
        in_specs=[
            pl.BlockSpec((TILE, H), lambda i: (i, 0)),
            pl.BlockSpec((N_EXPERTS, H), lambda i: (0, 0)),
        ],
        out_specs=[
            pl.BlockSpec((N_EXPERTS, TILE), lambda i: (0, i)),
            pl.BlockSpec((K, TILE), lambda i: (0, i)),
            pl.BlockSpec((K, TILE), lambda i: (0, i)),
        ],
        out_shape=[
            jax.ShapeDtypeStruct((N_EXPERTS, T), jnp.float32),
            jax.ShapeDtypeStruct((K, T), jnp.float32),
            jax.ShapeDtypeStruct((K, T), jnp.int32),
        ],
        compiler_params=pltpu.CompilerParams(
            dimension_semantics=("parallel",),
        ),
    )(x, gate_weight.astype(hidden_states.dtype))

    return (
        probs_t.T.reshape(B, S, K),
        idx_t.T.reshape(B, S, K),
        logits_t.T.reshape(B, S, N_EXPERTS),
    )
